# Initial kernel scaffold; baseline (speedup 1.0000x reference)
import jax
import jax.numpy as jnp
from jax.experimental import pallas as pl


def kernel(x, codebook):
    raise NotImplementedError("write your pallas kernel here")


# speedup vs baseline: 13.3201x; 13.3201x over previous
import math

import jax
import jax.numpy as jnp
from jax.experimental import pallas as pl
from jax.experimental.pallas import tpu as pltpu

N = 4
C = 64
S = 512
P = 196
K = 16
_EXPW = [math.exp(-k) for k in range(K)]


def _tc_body(x_ref, cb_ref, xq_ref, idx_ref, lcb_ref, lcm_ref):
    cb = cb_ref[...]
    cb2 = jnp.sum(cb * cb, axis=1, keepdims=True)
    iota_s = jax.lax.broadcasted_iota(jnp.int32, (S, P), 0)
    lcb = jnp.float32(0.0)
    lcm = jnp.float32(0.0)
    for n in range(N):
        xn = x_ref[n]
        xn2 = jnp.sum(xn * xn, axis=0, keepdims=True)
        dot = jax.lax.dot_general(cb, xn, (((1,), (0,)), ((), ())),
                                  precision=jax.lax.Precision.HIGHEST,
                                  preferred_element_type=jnp.float32)
        d = cb2 - 2.0 * dot + xn2
        m = jnp.min(d, axis=0, keepdims=True)
        fidx = jnp.min(jnp.where(d == m, iota_s, S), axis=0, keepdims=True)
        idx_ref[pl.ds(n, 1), :] = fidx
        lcm = lcm + jnp.sum(m)
        lcb = lcb + jnp.sum(m)
        dw = jnp.where(iota_s == fidx, jnp.inf, d)
        for k in range(1, K):
            mk = jnp.min(dw, axis=0, keepdims=True)
            lcb = lcb + jnp.float32(_EXPW[k]) * jnp.sum(mk)
            fk = jnp.min(jnp.where(dw == mk, iota_s, S), axis=0, keepdims=True)
            dw = jnp.where(iota_s == fk, jnp.inf, dw)
        onehot = (iota_s == fidx).astype(jnp.float32)
        xq = jax.lax.dot_general(cb, onehot, (((0,), (0,)), ((), ())),
                                 precision=jax.lax.Precision.HIGHEST,
                                 preferred_element_type=jnp.float32)
        xq_ref[n] = xq
    lcb_ref[0, 0] = lcb / jnp.float32(N * S * P)
    lcm_ref[0, 0] = lcm / jnp.float32(N * C * P)


def kernel(x, codebook):
    x3 = x.reshape(N, C, P)
    xq, idx, lcb, lcm = pl.pallas_call(
        _tc_body,
        out_shape=(
            jax.ShapeDtypeStruct((N, C, P), jnp.float32),
            jax.ShapeDtypeStruct((N, P), jnp.int32),
            jax.ShapeDtypeStruct((1, 1), jnp.float32),
            jax.ShapeDtypeStruct((1, 1), jnp.float32),
        ),
        out_specs=(
            pl.BlockSpec(memory_space=pltpu.VMEM),
            pl.BlockSpec(memory_space=pltpu.VMEM),
            pl.BlockSpec(memory_space=pltpu.SMEM),
            pl.BlockSpec(memory_space=pltpu.SMEM),
        ),
        in_specs=(
            pl.BlockSpec(memory_space=pltpu.VMEM),
            pl.BlockSpec(memory_space=pltpu.VMEM),
        ),
    )(x3, codebook)
    output = xq.reshape(x.shape)
    return (output, lcb[0, 0], lcm[0, 0], idx.reshape(N, 14, 14))
